# baseline (speedup 1.0000x reference)
import jax
import jax.numpy as jnp
from jax.experimental import pallas as pl
from jax.experimental.pallas import tpu as pltpu

BM = 512

_f32 = jnp.float32
_bf16 = jnp.bfloat16


def _dot(a, b, dn=None):
    if dn is None:
        dn = (((a.ndim - 1,), (0,)), ((), ()))
    return jax.lax.dot_general(a, b, dn, preferred_element_type=_f32)


def _xw1_kernel(x_hbm, w1_ref, o_ref, xv, sem):
    cp = pltpu.make_async_copy(x_hbm, xv, sem)
    cp.start()
    cp.wait()
    for s in range(4):
        sl = pl.ds(s * 1024, 1024)
        o_ref[sl, :] = _dot(xv[sl, :].astype(_bf16),
                            w1_ref[...]).astype(_bf16)


def _pass1_kernel(a_ref, p_ref, b1_ref, h_ref):
    acc = _dot(a_ref[...].astype(_bf16), p_ref[...])
    h_ref[...] = jax.nn.relu(acc + b1_ref[...]).astype(_bf16)


def _pass2_kernel(a_ref, h_ref, w2_ref, b2_ref, wd_ref, bd_ref, wb_ref,
                  z_ref, hd_ref, u_ref):
    r = _dot(a_ref[...].astype(_bf16), h_ref[...])
    z = _dot(r.astype(_bf16), w2_ref[...]) + b2_ref[...]
    z_ref[...] = z
    hd = _dot(z.astype(_bf16), wd_ref[...]) + bd_ref[...]
    hdb = hd.astype(_bf16)
    hd_ref[...] = hdb
    u_ref[...] = _dot(hdb, wb_ref[...]).astype(_bf16)


def _decx_kernel(z_ref, wx1_ref, bx1_ref, wx2_ref, bx2_ref,
                 wx3_ref, bx3_ref, wx4_ref, bx4_ref, xo_hbm, xov, sem):
    for s in range(4):
        sl = pl.ds(s * 1024, 1024)
        zb = z_ref[sl, :].astype(_bf16)
        hx = jax.nn.relu(_dot(zb, wx1_ref[...]) + bx1_ref[...])
        hx = jax.nn.relu(_dot(hx.astype(_bf16), wx2_ref[...]) + bx2_ref[...])
        hx = jax.nn.relu(_dot(hx.astype(_bf16), wx3_ref[...]) + bx3_ref[...])
        xov[sl, :] = _dot(hx.astype(_bf16), wx4_ref[...]) + bx4_ref[...]
    cp = pltpu.make_async_copy(xov, xo_hbm, sem)
    cp.start()
    cp.wait()


def _aout_kernel(u_ref, hd_ref, o_ref):
    s = _dot(u_ref[...], hd_ref[...], dn=(((1,), (1,)), ((), ())))
    o_ref[...] = 0.5 * jnp.tanh(0.5 * s) + 0.5


def _row_spec(bm, ncols, bufs=None):
    del bufs
    return pl.BlockSpec((bm, ncols), lambda i: (i, 0))


def _full_spec(shape):
    nd = len(shape)
    return pl.BlockSpec(shape, lambda i: (0,) * nd)


def _padc(a, w):
    return jnp.pad(a, ((0, 0), (0, w - a.shape[1])))


@jax.jit
def kernel(x, adj_n, W1, b1, W2, b2, Wd, bd, Wb,
           Wx1, bx1, Wx2, bx2, Wx3, bx3, Wx4, bx4):
    n, in_dim = x.shape
    lat = W2.shape[1]

    w1p = _padc(W1, 128).astype(_bf16)
    b1p = jnp.pad(b1, (0, 8)).reshape(1, -1)
    w2p = _padc(jnp.pad(W2, ((0, 8), (0, 0))), 128).astype(_bf16)
    b2p = jnp.pad(b2, (0, 108)).reshape(1, -1)
    wdp = _padc(jnp.pad(Wd, ((0, 108), (0, 0))), 128).astype(_bf16)
    bdp = jnp.pad(bd, (0, 64)).reshape(1, -1)
    wbp = _padc(jnp.pad(Wb, ((0, 64), (0, 0))), 128).astype(_bf16)
    wx1p = jnp.pad(Wx1, ((0, 108), (0, 0))).astype(_bf16)
    wx2b = Wx2.astype(_bf16)
    wx3b = Wx3.astype(_bf16)
    wx4b = Wx4.astype(_bf16)
    bx1r = bx1.reshape(1, -1)
    bx2r = bx2.reshape(1, -1)
    bx3r = bx3.reshape(1, -1)
    bx4r = bx4.reshape(1, -1)

    p = pl.pallas_call(
        _xw1_kernel,
        in_specs=[pl.BlockSpec(memory_space=pl.ANY),
                  pl.BlockSpec(memory_space=pltpu.MemorySpace.VMEM)],
        out_specs=pl.BlockSpec(memory_space=pltpu.MemorySpace.VMEM),
        out_shape=jax.ShapeDtypeStruct((n, 128), _bf16),
        scratch_shapes=[pltpu.VMEM((n, in_dim), _f32),
                        pltpu.SemaphoreType.DMA],
    )(x, w1p)

    h = pl.pallas_call(
        _pass1_kernel,
        grid=(n // BM,),
        in_specs=[_row_spec(BM, n, bufs=4), _full_spec(p.shape),
                  _full_spec((1, 128))],
        out_specs=_row_spec(BM, 128),
        out_shape=jax.ShapeDtypeStruct((n, 128), _bf16),
    )(adj_n, p, b1p)

    zp, hd, u = pl.pallas_call(
        _pass2_kernel,
        grid=(n // BM,),
        in_specs=[
            _row_spec(BM, n, bufs=4),
            _full_spec(h.shape),
            _full_spec(w2p.shape), _full_spec(b2p.shape),
            _full_spec(wdp.shape), _full_spec(bdp.shape),
            _full_spec(wbp.shape),
        ],
        out_specs=[
            _row_spec(BM, 128),
            _row_spec(BM, 128),
            _row_spec(BM, 128),
        ],
        out_shape=[
            jax.ShapeDtypeStruct((n, 128), _f32),
            jax.ShapeDtypeStruct((n, 128), _bf16),
            jax.ShapeDtypeStruct((n, 128), _bf16),
        ],
    )(adj_n, h, w2p, b2p, wdp, bdp, wbp)

    x_out = pl.pallas_call(
        _decx_kernel,
        in_specs=[
            pl.BlockSpec(memory_space=pltpu.MemorySpace.VMEM),
            pl.BlockSpec(memory_space=pltpu.MemorySpace.VMEM),
            pl.BlockSpec(memory_space=pltpu.MemorySpace.VMEM),
            pl.BlockSpec(memory_space=pltpu.MemorySpace.VMEM),
            pl.BlockSpec(memory_space=pltpu.MemorySpace.VMEM),
            pl.BlockSpec(memory_space=pltpu.MemorySpace.VMEM),
            pl.BlockSpec(memory_space=pltpu.MemorySpace.VMEM),
            pl.BlockSpec(memory_space=pltpu.MemorySpace.VMEM),
            pl.BlockSpec(memory_space=pltpu.MemorySpace.VMEM),
        ],
        out_specs=pl.BlockSpec(memory_space=pl.ANY),
        out_shape=jax.ShapeDtypeStruct((n, in_dim), _f32),
        scratch_shapes=[pltpu.VMEM((n, in_dim), _f32),
                        pltpu.SemaphoreType.DMA],
    )(zp, wx1p, bx1r, wx2b, bx2r, wx3b, bx3r, wx4b, bx4r)

    a_out = pl.pallas_call(
        _aout_kernel,
        grid=(n // BM,),
        in_specs=[_row_spec(BM, 128), _full_spec(hd.shape)],
        out_specs=_row_spec(BM, n, bufs=4),
        out_shape=jax.ShapeDtypeStruct((n, n), _f32),
    )(u, hd)

    return zp[:, :lat], x_out, a_out


# speedup vs baseline: 1.0833x; 1.0833x over previous
import jax
import jax.numpy as jnp
from jax.experimental import pallas as pl

BM = 512
BMX = 1024


def _mm(a, b, dn=None):
    if dn is None:
        dn = (((a.ndim - 1,), (0,)), ((), ()))
    return jax.lax.dot_general(
        a.astype(jnp.bfloat16), b.astype(jnp.bfloat16), dn,
        preferred_element_type=jnp.float32)


def _xw1_kernel(x_ref, w1_ref, o_ref):
    o_ref[...] = _mm(x_ref[...], w1_ref[...])


def _gcn1_kernel(al_ref, ar_ref, pl_ref, pr_ref, b1_ref, o_ref):
    acc = _mm(al_ref[...], pl_ref[...]) + _mm(ar_ref[...], pr_ref[...])
    o_ref[...] = jax.nn.relu(acc + b1_ref[...])


def _gcn2_kernel(al_ref, ar_ref, hl_ref, hr_ref, w2_ref, b2_ref,
                 wd_ref, bd_ref, wb_ref, z_ref, hd_ref, u_ref):
    r = _mm(al_ref[...], hl_ref[...]) + _mm(ar_ref[...], hr_ref[...])
    z = _mm(r, w2_ref[...]) + b2_ref[...]
    z_ref[...] = z
    hd = _mm(z, wd_ref[...]) + bd_ref[...]
    hd_ref[...] = hd
    u_ref[...] = _mm(hd, wb_ref[...])


def _decx_kernel(z_ref, wx1_ref, bx1_ref, wx2_ref, bx2_ref,
                 wx3_ref, bx3_ref, wx4_ref, bx4_ref, xo_ref):
    hx = jax.nn.relu(_mm(z_ref[...], wx1_ref[...]) + bx1_ref[...])
    hx = jax.nn.relu(_mm(hx, wx2_ref[...]) + bx2_ref[...])
    hx = jax.nn.relu(_mm(hx, wx3_ref[...]) + bx3_ref[...])
    xo_ref[...] = _mm(hx, wx4_ref[...]) + bx4_ref[...]


def _aout_kernel(u_ref, hd_ref, o_ref):
    s = _mm(u_ref[...], hd_ref[...], dn=(((1,), (1,)), ((), ())))
    o_ref[...] = 0.5 * jnp.tanh(0.5 * s) + 0.5


def _row_spec(bm, ncols):
    return pl.BlockSpec((bm, ncols), lambda i: (i, 0))


def _full_spec(shape):
    nd = len(shape)
    return pl.BlockSpec(shape, lambda i: (0,) * nd)


@jax.jit
def kernel(x, adj_n, W1, b1, W2, b2, Wd, bd, Wb,
           Wx1, bx1, Wx2, bx2, Wx3, bx3, Wx4, bx4):
    n, in_dim = x.shape
    hid = W1.shape[1]
    lat = W2.shape[1]
    adj_dim = Wd.shape[1]
    grid = (n // BM,)
    nh = n // 2
    f32 = jnp.float32

    b1r = b1.reshape(1, -1)
    b2r = b2.reshape(1, -1)
    bdr = bd.reshape(1, -1)
    bx1r = bx1.reshape(1, -1)
    bx2r = bx2.reshape(1, -1)
    bx3r = bx3.reshape(1, -1)
    bx4r = bx4.reshape(1, -1)

    a_strip_l = pl.BlockSpec((BM, nh), lambda i: (i, 0))
    a_strip_r = pl.BlockSpec((BM, nh), lambda i: (i, 1))

    p = pl.pallas_call(
        _xw1_kernel,
        grid=grid,
        in_specs=[_row_spec(BM, in_dim), _full_spec(W1.shape)],
        out_specs=_row_spec(BM, hid),
        out_shape=jax.ShapeDtypeStruct((n, hid), f32),
    )(x, W1)

    h = pl.pallas_call(
        _gcn1_kernel,
        grid=grid,
        in_specs=[
            a_strip_l, a_strip_r,
            pl.BlockSpec((nh, hid), lambda i: (0, 0)),
            pl.BlockSpec((nh, hid), lambda i: (1, 0)),
            _full_spec(b1r.shape),
        ],
        out_specs=_row_spec(BM, hid),
        out_shape=jax.ShapeDtypeStruct((n, hid), f32),
    )(adj_n, adj_n, p, p, b1r)

    z, hd, u = pl.pallas_call(
        _gcn2_kernel,
        grid=grid,
        in_specs=[
            a_strip_l, a_strip_r,
            pl.BlockSpec((nh, hid), lambda i: (0, 0)),
            pl.BlockSpec((nh, hid), lambda i: (1, 0)),
            _full_spec(W2.shape), _full_spec(b2r.shape),
            _full_spec(Wd.shape), _full_spec(bdr.shape),
            _full_spec(Wb.shape),
        ],
        out_specs=[
            _row_spec(BM, lat),
            _row_spec(BM, adj_dim),
            _row_spec(BM, adj_dim),
        ],
        out_shape=[
            jax.ShapeDtypeStruct((n, lat), f32),
            jax.ShapeDtypeStruct((n, adj_dim), f32),
            jax.ShapeDtypeStruct((n, adj_dim), f32),
        ],
    )(adj_n, adj_n, h, h, W2, b2r, Wd, bdr, Wb)

    x_out = pl.pallas_call(
        _decx_kernel,
        grid=(n // BMX,),
        in_specs=[
            _row_spec(BMX, lat),
            _full_spec(Wx1.shape), _full_spec(bx1r.shape),
            _full_spec(Wx2.shape), _full_spec(bx2r.shape),
            _full_spec(Wx3.shape), _full_spec(bx3r.shape),
            _full_spec(Wx4.shape), _full_spec(bx4r.shape),
        ],
        out_specs=_row_spec(BMX, in_dim),
        out_shape=jax.ShapeDtypeStruct((n, in_dim), f32),
    )(z, Wx1, bx1r, Wx2, bx2r, Wx3, bx3r, Wx4, bx4r)

    a_out = pl.pallas_call(
        _aout_kernel,
        grid=grid,
        in_specs=[_row_spec(BM, adj_dim), _full_spec(hd.shape)],
        out_specs=_row_spec(BM, n),
        out_shape=jax.ShapeDtypeStruct((n, n), f32),
    )(u, hd)

    return z, x_out, a_out
